# Initial kernel scaffold; baseline (speedup 1.0000x reference)
import jax
import jax.numpy as jnp
from jax.experimental import pallas as pl


def kernel(x, edge_index, batch, W1, as1, ad1, b1, W2, as2, ad2, b2, W3, as3, ad3, b3, g1, be1, g2, be2):
    raise NotImplementedError("write your pallas kernel here")


# speedup vs baseline: 39.9274x; 39.9274x over previous
import functools

import jax
import jax.numpy as jnp
from jax import lax
from jax.experimental import pallas as pl
from jax.experimental.pallas import tpu as pltpu
from jax.experimental.pallas import tpu_sc as plsc

N = 10000
N_PAD = 10240
DUMMY = 10000
G = 400
E_TOT = 330000
NC, NS, L = 2, 16, 16
K = 128
STEPS = 81
E_PAD = NC * NS * K * STEPS
ROWS_PER_TILE = N_PAD // NS
CHUNK = 64


def _sc_gat_body(heads, ch, ws, tsrc, tdst, src, dst, acc_out,
                 acc, buf, dbuf, sidx, didx, bounce):
    d = heads * ch
    c = lax.axis_index("c")
    s = lax.axis_index("s")
    wid = c * NS + s

    for j in range(CHUNK):
        for k in range(ws // L):
            bounce[j, pl.ds(k * L, L)] = jnp.zeros((L,), jnp.float32)
    r0 = s * ROWS_PER_TILE
    @pl.loop(0, ROWS_PER_TILE // CHUNK)
    def _zero(j):
        pltpu.sync_copy(bounce, acc.at[pl.ds(r0 + j * CHUNK, CHUNK)])
    plsc.subcore_barrier()

    e_start = wid * (STEPS * K)
    _edge_scan(heads, ch, d, e_start, tsrc, tdst, src, dst, acc,
               buf, dbuf, sidx, didx)

    plsc.subcore_barrier()

    @pl.loop(0, ROWS_PER_TILE // CHUNK)
    def _out(j):
        rr = r0 + j * CHUNK
        pltpu.sync_copy(acc.at[pl.ds(rr, CHUNK)], bounce)
        pltpu.sync_copy(bounce, acc_out.at[c].at[pl.ds(rr, CHUNK)])


def _edge_scan(heads, ch, d, e_start, tsrc, tdst, src, dst, acc,
               buf, dbuf, sidx, didx):
    @pl.loop(0, STEPS)
    def _win(w):
        base = e_start + w * K
        pltpu.sync_copy(src.at[pl.ds(base, K)], sidx)
        pltpu.sync_copy(dst.at[pl.ds(base, K)], didx)
        pltpu.sync_copy(tsrc.at[sidx], buf)
        pltpu.sync_copy(tdst.at[didx], dbuf)

        @pl.loop(0, K)
        def _edge(e):
            iot = lax.broadcasted_iota(jnp.int32, (L,), 0)
            ddv = dbuf[e, pl.ds(0, L)]
            if d >= L:
                adv = buf[e, pl.ds(d, L)]
                sv = adv + ddv
                sv = jnp.maximum(sv, 0.2 * sv)
                pvv = jnp.exp(sv)
                for k in range(d // L):
                    pv = jnp.full((L,), pvv[(k * L) // ch], jnp.float32)
                    v = buf[e, pl.ds(k * L, L)]
                    buf[e, pl.ds(k * L, L)] = v * pv
                buf[e, pl.ds(d, L)] = jnp.where(iot < heads, pvv, 0.0)
            else:
                v = buf[e, pl.ds(0, L)]
                sc = v[d] + ddv[0]
                sc = jnp.maximum(sc, 0.2 * sc)
                pv = jnp.exp(jnp.full((L,), sc, jnp.float32))
                out = jnp.where(iot < d, v * pv,
                                jnp.where(iot == d, pv, 0.0))
                buf[e, pl.ds(0, L)] = out

        pltpu.sync_copy(buf, acc.at[didx], add=True)


def _sc_gat(heads, ch, ws, tsrc, tdst, src, dst):
    mesh = plsc.VectorSubcoreMesh(core_axis_name="c", subcore_axis_name="s")
    body = functools.partial(_sc_gat_body, heads, ch, ws)
    return pl.kernel(
        body,
        out_type=jax.ShapeDtypeStruct((NC, N_PAD, ws), jnp.float32),
        mesh=mesh,
        scratch_types=[
            pltpu.VMEM_SHARED((N_PAD, ws), jnp.float32),
            pltpu.VMEM((K, ws), jnp.float32),
            pltpu.VMEM((K, 16), jnp.float32),
            pltpu.VMEM((K,), jnp.int32),
            pltpu.VMEM((K,), jnp.int32),
            pltpu.VMEM((CHUNK, ws), jnp.float32),
        ],
        compiler_params=pltpu.CompilerParams(use_tc_tiling_on_sc=False),
        name=f"sc_gat_h{heads}",
    )(tsrc, tdst, src, dst)


def _tc1_body(x_ref, w_ref, asp_ref, adp_ref, tsrc_ref, tdst_ref):
    h = jnp.dot(x_ref[...], w_ref[...], preferred_element_type=jnp.float32, precision=lax.Precision.HIGHEST)
    a = jnp.dot(h, asp_ref[...], preferred_element_type=jnp.float32, precision=lax.Precision.HIGHEST)
    dd = jnp.dot(h, adp_ref[...], preferred_element_type=jnp.float32, precision=lax.Precision.HIGHEST)
    tsrc_ref[...] = jnp.concatenate([h, a], axis=1)
    tdst_ref[...] = dd


def _tc1(x_pad, w, asp, adp, d, pw):
    return pl.pallas_call(
        _tc1_body,
        out_shape=[jax.ShapeDtypeStruct((N_PAD, d + pw), jnp.float32),
                   jax.ShapeDtypeStruct((N_PAD, 16), jnp.float32)],
        compiler_params=pltpu.CompilerParams(
            vmem_limit_bytes=100 * 1024 * 1024),
        name="tc_proj1",
    )(x_pad, w, asp, adp)


def _merge_body(d_in, acc_ref, b_ref, g_ref, be_ref, w_ref, asp_ref,
                adp_ref, ee_ref, tsrc_ref, tdst_ref):
    num = acc_ref[0, :, 0:d_in] + acc_ref[1, :, 0:d_in]
    den16 = acc_ref[0, :, d_in:d_in + 16] + acc_ref[1, :, d_in:d_in + 16]
    den = jnp.dot(den16, ee_ref[...], preferred_element_type=jnp.float32, precision=lax.Precision.HIGHEST)
    o = num / (den + 1e-16) + b_ref[...]
    o = jnp.maximum(o, 0.0)
    m_row = (lax.broadcasted_iota(jnp.int32, (N_PAD, 1), 0) < N)
    mask = m_row.astype(jnp.float32)
    om = o * mask
    mu = jnp.sum(om, axis=0, keepdims=True) / N
    var = jnp.sum(((o - mu) * mask) ** 2, axis=0, keepdims=True) / N
    y = (o - mu) / jnp.sqrt(var + 1e-5) * g_ref[...] + be_ref[...]
    h = jnp.dot(y, w_ref[...], preferred_element_type=jnp.float32, precision=lax.Precision.HIGHEST)
    a = jnp.dot(h, asp_ref[...], preferred_element_type=jnp.float32, precision=lax.Precision.HIGHEST)
    dd = jnp.dot(h, adp_ref[...], preferred_element_type=jnp.float32, precision=lax.Precision.HIGHEST)
    tsrc_ref[...] = jnp.concatenate([h, a], axis=1)
    tdst_ref[...] = dd


def _tc_merge(acc, b, gam, bet, w, asp, adp, ee, d_in, d_out, pw):
    return pl.pallas_call(
        functools.partial(_merge_body, d_in),
        out_shape=[jax.ShapeDtypeStruct((N_PAD, d_out + pw), jnp.float32),
                   jax.ShapeDtypeStruct((N_PAD, 16), jnp.float32)],
        compiler_params=pltpu.CompilerParams(
            vmem_limit_bytes=100 * 1024 * 1024),
        name="tc_merge",
    )(acc, b, gam, bet, w, asp, adp, ee)


def _final_body(acc_ref, b_ref, batch_ref, out_ref):
    num = acc_ref[0, :, 0:2] + acc_ref[1, :, 0:2]
    den = acc_ref[0, :, 2:3] + acc_ref[1, :, 2:3]
    o = num / (den + 1e-16) + b_ref[...]
    bid = batch_ref[...]
    gi = lax.broadcasted_iota(jnp.int32, (G, N_PAD), 0)
    mm = (gi == bid).astype(jnp.float32)
    cnt = jnp.sum(mm, axis=1, keepdims=True)
    sums = jnp.dot(mm, o, preferred_element_type=jnp.float32, precision=lax.Precision.HIGHEST)
    out_ref[...] = sums / jnp.maximum(cnt, 1.0)


def _tc_final(acc, b, batch_pad):
    return pl.pallas_call(
        _final_body,
        out_shape=jax.ShapeDtypeStruct((G, 2), jnp.float32),
        compiler_params=pltpu.CompilerParams(
            vmem_limit_bytes=100 * 1024 * 1024),
        name="tc_final",
    )(acc, b, batch_pad)


def _att_proj(att, d, pw):
    heads = att.shape[0]
    return (jnp.eye(heads, pw, dtype=jnp.float32)[:, None, :]
            * att[:, :, None]).reshape(d, pw)


def kernel(x, edge_index, batch, W1, as1, ad1, b1, W2, as2, ad2, b2,
           W3, as3, ad3, b3, g1, be1, g2, be2):
    x_pad = jnp.zeros((N_PAD, x.shape[1]), jnp.float32).at[:N].set(x)
    loops = jnp.arange(N, dtype=jnp.int32)
    src = jnp.concatenate([edge_index[0], loops,
                           jnp.zeros((E_PAD - E_TOT,), jnp.int32)])
    dst = jnp.concatenate([edge_index[1], loops,
                           jnp.full((E_PAD - E_TOT,), DUMMY, jnp.int32)])
    batch_pad = jnp.concatenate(
        [batch, jnp.full((N_PAD - N,), G + 1, jnp.int32)]).reshape(1, N_PAD)

    as1p, ad1p = _att_proj(as1, 64, 16), _att_proj(ad1, 64, 16)
    as2p, ad2p = _att_proj(as2, 64, 16), _att_proj(ad2, 64, 16)
    as3p, ad3p = _att_proj(as3, 2, 14), _att_proj(ad3, 2, 16)
    ee1 = (jnp.arange(16)[:, None] == (jnp.arange(64)[None, :] // 16)
           ).astype(jnp.float32)
    ee2 = (jnp.arange(16)[:, None] == (jnp.arange(64)[None, :] // 32)
           ).astype(jnp.float32)
    b1r, g1r, be1r = b1.reshape(1, 64), g1.reshape(1, 64), be1.reshape(1, 64)
    b2r, g2r, be2r = b2.reshape(1, 64), g2.reshape(1, 64), be2.reshape(1, 64)
    b3r = b3.reshape(1, 2)

    tsrc1, tdst1 = _tc1(x_pad, W1, as1p, ad1p, 64, 16)
    acc1 = _sc_gat(4, 16, 80, tsrc1, tdst1, src, dst)
    tsrc2, tdst2 = _tc_merge(acc1, b1r, g1r, be1r, W2, as2p, ad2p, ee1,
                             64, 64, 16)
    acc2 = _sc_gat(2, 32, 80, tsrc2, tdst2, src, dst)
    tsrc3, tdst3 = _tc_merge(acc2, b2r, g2r, be2r, W3, as3p, ad3p, ee2,
                             64, 2, 14)
    acc3 = _sc_gat(1, 2, 16, tsrc3, tdst3, src, dst)
    return _tc_final(acc3, b3r, batch_pad)
